# baseline (speedup 1.0000x reference)
import functools

import jax
import jax.numpy as jnp
from jax import lax
from jax.experimental import pallas as pl
from jax.experimental.pallas import tpu as pltpu
from jax.experimental.pallas import tpu_sc as plsc

C = 256
F32 = jnp.float32
BF16 = jnp.bfloat16
I32 = jnp.int32

NC = 2
NS = 16
NW = NC * NS
L = 16


def _full(shape):
    return pl.BlockSpec(shape, lambda b: tuple(0 for _ in shape))


def _make_sc_knn(B, N0, N1, N2):
    mesh = plsc.VectorSubcoreMesh(core_axis_name="c", subcore_axis_name="s")
    R = 64
    PITCH = L + 1
    q1 = N1 // 2
    q0 = N0 // 2
    nc1 = q1 // R
    nc0 = q0 // R

    def body(pwd_hbm, idx12_hbm, idx01_hbm,
             pwd_v0, pwd_v1, pwd_v2, idx_v0, idx_v1, idx_v2, vbuf, ibuf,
             sp0, sp1, sp2, so0, so1, so2):
        cix = lax.axis_index("c")
        six = lax.axis_index("s")
        w = cix * NS + six
        b = w // 2
        half = w % 2
        lane = lax.broadcasted_iota(I32, (L,), 0)
        pwd_v = (pwd_v0, pwd_v1, pwd_v2)
        idx_v = (idx_v0, idx_v1, idx_v2)
        sp = (sp0, sp1, sp2)
        so = (so0, so1, so2)

        def argmin_chunk(src_ref, k_cols, iv):
            def group_body(g, carry):
                rbase = g * L
                for rr in range(L):
                    r = rbase + rr
                    pairs = []
                    for c16 in range(k_cols // L):
                        v = src_ref[r, pl.ds(c16 * L, L)]
                        pairs.append((v, lane + (c16 * L)))
                    while len(pairs) > 1:
                        nxt = []
                        for k in range(0, len(pairs) - 1, 2):
                            va, ia = pairs[k]
                            vb, ib = pairs[k + 1]
                            mlt = vb < va
                            nxt.append((jnp.where(mlt, vb, va),
                                        jnp.where(mlt, ib, ia)))
                        if len(pairs) % 2:
                            nxt.append(pairs[-1])
                        pairs = nxt
                    v, i = pairs[0]
                    vbuf[pl.ds(rr * PITCH, L)] = v
                    ibuf[pl.ds(rr * PITCH, L)] = i
                col = lane * PITCH
                bv = plsc.load_gather(vbuf, [col])
                bi = plsc.load_gather(ibuf, [col])
                for cc in range(1, L):
                    pv = plsc.load_gather(vbuf, [col + cc])
                    pi = plsc.load_gather(ibuf, [col + cc])
                    better = (pv < bv) | ((pv == bv) & (pi < bi))
                    bv = jnp.where(better, pv, bv)
                    bi = jnp.where(better, pi, bi)
                iv[pl.ds(rbase, L)] = bi
                return carry

            lax.fori_loop(0, R // L, group_body, 0)

        row1 = half * q1

        def pwd12_src(ch):
            return pwd_hbm.at[b, pl.ds(row1 + ch * R, R), pl.ds(0, N2)]

        def pwd12_dst(q):
            return pwd_v[q].at[:, pl.ds(0, N2)]

        def idx12_dst(ch):
            return idx12_hbm.at[pl.ds(b * N1 + row1 + ch * R, R)]

        pltpu.async_copy(pwd12_src(0), pwd12_dst(0), sp0)
        pltpu.async_copy(pwd12_src(1), pwd12_dst(1), sp1)
        for ch in range(nc1):
            q = ch % 2
            pltpu.make_async_copy(pwd12_src(ch), pwd12_dst(q), sp[q]).wait()
            if ch >= 2:
                pltpu.make_async_copy(idx_v[q], idx12_dst(ch - 2),
                                      so[q]).wait()
            argmin_chunk(pwd_v[q], N2, idx_v[q])
            if ch + 2 < nc1:
                pltpu.async_copy(pwd12_src(ch + 2), pwd12_dst(q), sp[q])
            pltpu.async_copy(idx_v[q], idx12_dst(ch), so[q])
        pltpu.make_async_copy(idx_v[0], idx12_dst(nc1 - 2), so[0]).wait()
        pltpu.make_async_copy(idx_v[1], idx12_dst(nc1 - 1), so[1]).wait()

        row0w = half * q0

        def pwd_src(ch):
            return pwd_hbm.at[b, pl.ds(row0w + ch * R, R), pl.ds(0, N1)]

        def out_dst(ch):
            return idx01_hbm.at[pl.ds(b * N0 + row0w + ch * R, R)]

        pltpu.async_copy(pwd_src(0), pwd_v[0], sp[0])
        pltpu.async_copy(pwd_src(1), pwd_v[1], sp[1])
        pltpu.async_copy(pwd_src(2), pwd_v[2], sp[2])

        def tri_body(t, carry):
            for s in (0, 1, 2):
                ch = 3 * t + s
                pltpu.make_async_copy(pwd_src(ch), pwd_v[s], sp[s]).wait()

                @pl.when(t > 0)
                def _():
                    pltpu.make_async_copy(idx_v[s], out_dst(ch - 3),
                                          so[s]).wait()
                argmin_chunk(pwd_v[s], N1, idx_v[s])

                @pl.when(ch + 3 < nc0)
                def _():
                    pltpu.async_copy(pwd_src(ch + 3), pwd_v[s], sp[s])
                pltpu.async_copy(idx_v[s], out_dst(ch), so[s])
            return carry

        lax.fori_loop(0, nc0 // 3, tri_body, 0)
        pltpu.make_async_copy(pwd_src(nc0 - 1), pwd_v[0], sp[0]).wait()
        pltpu.make_async_copy(idx_v[0], out_dst(nc0 - 4), so[0]).wait()
        argmin_chunk(pwd_v[0], N1, idx_v[0])
        pltpu.async_copy(idx_v[0], out_dst(nc0 - 1), so[0])
        pltpu.make_async_copy(idx_v[0], out_dst(nc0 - 1), so[0]).wait()
        pltpu.make_async_copy(idx_v[1], out_dst(nc0 - 3), so[1]).wait()
        pltpu.make_async_copy(idx_v[2], out_dst(nc0 - 2), so[2]).wait()

    return pl.kernel(
        body,
        out_type=[
            jax.ShapeDtypeStruct((B * N1,), I32),
            jax.ShapeDtypeStruct((B * N0,), I32),
        ],
        mesh=mesh,
        scratch_types=[
            pltpu.VMEM((R, N1), F32),
            pltpu.VMEM((R, N1), F32),
            pltpu.VMEM((R, N1), F32),
            pltpu.VMEM((R,), I32),
            pltpu.VMEM((R,), I32),
            pltpu.VMEM((R,), I32),
            pltpu.VMEM((L * PITCH,), F32),
            pltpu.VMEM((L * PITCH,), I32),
            pltpu.SemaphoreType.DMA,
            pltpu.SemaphoreType.DMA,
            pltpu.SemaphoreType.DMA,
            pltpu.SemaphoreType.DMA,
            pltpu.SemaphoreType.DMA,
            pltpu.SemaphoreType.DMA,
        ],
        compiler_params=pltpu.CompilerParams(needs_layout_passes=False),
    )


def _fused_body(x0, x2, x1, i12, i01, W_all, b_all, Wp2a, Wp2b, W2a3, cvec2,
                Wp1a, M1, A1, c1, Wp0a, M0, A0, c0,
                feat2_o, feat1_o, feat0_o):
    x0a = x0[0, :128]
    x0b = x0[0, :512]
    f2 = jnp.dot(x0a, W_all[...], preferred_element_type=F32) + b_all[...]
    cls2 = jnp.max(f2, axis=0, keepdims=True)
    cls_t = jnp.dot(cls2, Wp2a[...], preferred_element_type=F32)
    feat2 = (cls_t
             + jnp.dot(x2[0], W2a3[...], preferred_element_type=F32)
             + jnp.dot(f2.astype(BF16), Wp2b[...].astype(BF16),
                       preferred_element_type=F32)
             + cvec2[...])
    feat2_o[0] = feat2
    T1 = (jnp.dot(feat2.astype(BF16), Wp1a[...].astype(BF16),
                  preferred_element_type=F32)
          - jnp.dot(x2[0], M1[...], preferred_element_type=F32))

    idx12 = i12[0]
    iota2 = lax.broadcasted_iota(I32, (4, 128, 128), 2)
    oh1 = (iota2 == idx12[:, :, None]).astype(F32).reshape(512, 128)
    G1 = jnp.dot(oh1, T1, preferred_element_type=F32)
    feat1 = (G1
             + jnp.dot(x1[0], M1[...], preferred_element_type=F32)
             + jnp.dot(x0b, A1[...], preferred_element_type=F32)
             + c1[...])
    feat1_o[0] = feat1
    T0 = (jnp.dot(feat1.astype(BF16), Wp0a[...].astype(BF16),
                  preferred_element_type=F32)
          - jnp.dot(x1[0], M0[...], preferred_element_type=F32))

    idx01 = i01[0]
    iota0 = lax.broadcasted_iota(I32, (16, 128, 512), 2)
    oh0 = (iota0 == idx01[:, :, None]).astype(BF16).reshape(2048, 512)
    G0 = jnp.dot(oh0, T0.astype(BF16), preferred_element_type=F32)
    feat0_o[0] = (G0
                  + jnp.dot(x0[0], A0[...], preferred_element_type=F32)
                  + c0[...])


def kernel(xyz0, xyz1, xyz2, pwd, W_all, b_all, W2, b2, W1, b1, W0, b0,
           Wp2, bp2, Wp1, bp1, Wp0, bp0):
    B, N0, _ = xyz0.shape
    N1 = xyz1.shape[1]
    N2 = xyz2.shape[1]

    Wp2a, Wp2b = Wp2[:C], Wp2[C:]
    Wp1a, Wp1b = Wp1[:C], Wp1[C:]
    Wp0a, Wp0b = Wp0[:C], Wp0[C:]
    W2a3 = W2 @ Wp2a
    cvec2 = (b2 @ Wp2a + bp2)[None, :]
    M1 = W1 @ Wp1a
    A1 = W_all @ Wp1b
    c1 = (b1 @ Wp1a + b_all @ Wp1b + bp1)[None, :]
    M0 = W0 @ Wp0a
    A0 = M0 + W_all @ Wp0b
    c0 = (b0 @ Wp0a + b_all @ Wp0b + bp0)[None, :]
    b_all2 = b_all[None, :]

    idx12f, idx01f = _make_sc_knn(B, N0, N1, N2)(pwd)
    idx12 = idx12f.reshape(B, N1 // 128, 128)
    idx01 = idx01f.reshape(B, N0 // 128, 128)

    feat2, feat1, feat0 = pl.pallas_call(
        _fused_body,
        grid=(B,),
        in_specs=[
            pl.BlockSpec((1, N0, 3), lambda b: (b, 0, 0)),
            pl.BlockSpec((1, N2, 3), lambda b: (b, 0, 0)),
            pl.BlockSpec((1, N1, 3), lambda b: (b, 0, 0)),
            pl.BlockSpec((1, N1 // 128, 128), lambda b: (b, 0, 0)),
            pl.BlockSpec((1, N0 // 128, 128), lambda b: (b, 0, 0)),
            _full((3, C)), _full((1, C)), _full((C, C)), _full((C, C)),
            _full((3, C)), _full((1, C)), _full((C, C)), _full((3, C)),
            _full((3, C)), _full((1, C)), _full((C, C)), _full((3, C)),
            _full((3, C)), _full((1, C)),
        ],
        out_specs=[
            pl.BlockSpec((1, N2, C), lambda b: (b, 0, 0)),
            pl.BlockSpec((1, N1, C), lambda b: (b, 0, 0)),
            pl.BlockSpec((1, N0, C), lambda b: (b, 0, 0)),
        ],
        out_shape=[
            jax.ShapeDtypeStruct((B, N2, C), F32),
            jax.ShapeDtypeStruct((B, N1, C), F32),
            jax.ShapeDtypeStruct((B, N0, C), F32),
        ],
    )(xyz0, xyz2, xyz1, idx12, idx01, W_all, b_all2, Wp2a, Wp2b, W2a3,
      cvec2, Wp1a, M1, A1, c1, Wp0a, M0, A0, c0)

    return (feat2, feat1, feat0)


# speedup vs baseline: 1.0282x; 1.0282x over previous
import functools

import jax
import jax.numpy as jnp
from jax import lax
from jax.experimental import pallas as pl
from jax.experimental.pallas import tpu as pltpu
from jax.experimental.pallas import tpu_sc as plsc

C = 256
F32 = jnp.float32
BF16 = jnp.bfloat16
I32 = jnp.int32

NC = 2
NS = 16
NW = NC * NS
L = 16


def _full(shape):
    return pl.BlockSpec(shape, lambda b: tuple(0 for _ in shape))


def _make_sc_knn(B, N0, N1, N2):
    mesh = plsc.VectorSubcoreMesh(core_axis_name="c", subcore_axis_name="s")
    R = 64
    PITCH = L + 1
    q1 = N1 // 2
    q0 = N0 // 2
    nc1 = q1 // R
    nc0 = q0 // R

    def body(pwd_hbm, idx12_hbm, idx01_hbm,
             pwd_v0, pwd_v1, idx_v0, idx_v1, vbuf, ibuf,
             sp0, sp1, so0, so1):
        cix = lax.axis_index("c")
        six = lax.axis_index("s")
        w = cix * NS + six
        b = w // 2
        half = w % 2
        lane = lax.broadcasted_iota(I32, (L,), 0)
        pwd_v = (pwd_v0, pwd_v1)
        idx_v = (idx_v0, idx_v1)
        sp = (sp0, sp1)
        so = (so0, so1)

        def argmin_chunk(src_ref, k_cols, iv):
            def group_body(g, carry):
                rbase = g * L
                for rr in range(L):
                    r = rbase + rr
                    pairs = []
                    for c16 in range(k_cols // L):
                        v = src_ref[r, pl.ds(c16 * L, L)]
                        pairs.append((v, lane + (c16 * L)))
                    while len(pairs) > 1:
                        nxt = []
                        for k in range(0, len(pairs) - 1, 2):
                            va, ia = pairs[k]
                            vb, ib = pairs[k + 1]
                            mlt = vb < va
                            nxt.append((jnp.where(mlt, vb, va),
                                        jnp.where(mlt, ib, ia)))
                        if len(pairs) % 2:
                            nxt.append(pairs[-1])
                        pairs = nxt
                    v, i = pairs[0]
                    vbuf[pl.ds(rr * PITCH, L)] = v
                    ibuf[pl.ds(rr * PITCH, L)] = i
                col = lane * PITCH
                bv = plsc.load_gather(vbuf, [col])
                bi = plsc.load_gather(ibuf, [col])
                for cc in range(1, L):
                    pv = plsc.load_gather(vbuf, [col + cc])
                    pi = plsc.load_gather(ibuf, [col + cc])
                    better = (pv < bv) | ((pv == bv) & (pi < bi))
                    bv = jnp.where(better, pv, bv)
                    bi = jnp.where(better, pi, bi)
                iv[pl.ds(rbase, L)] = bi
                return carry

            lax.fori_loop(0, R // L, group_body, 0)

        row1 = half * q1

        def pwd12_src(ch):
            return pwd_hbm.at[b, pl.ds(row1 + ch * R, R), pl.ds(0, N2)]

        def pwd12_dst(q):
            return pwd_v[q].at[:, pl.ds(0, N2)]

        def idx12_dst(ch):
            return idx12_hbm.at[pl.ds(b * N1 + row1 + ch * R, R)]

        pltpu.async_copy(pwd12_src(0), pwd12_dst(0), sp0)
        pltpu.async_copy(pwd12_src(1), pwd12_dst(1), sp1)
        for ch in range(nc1):
            q = ch % 2
            pltpu.make_async_copy(pwd12_src(ch), pwd12_dst(q), sp[q]).wait()
            if ch >= 2:
                pltpu.make_async_copy(idx_v[q], idx12_dst(ch - 2),
                                      so[q]).wait()
            argmin_chunk(pwd_v[q], N2, idx_v[q])
            if ch + 2 < nc1:
                pltpu.async_copy(pwd12_src(ch + 2), pwd12_dst(q), sp[q])
            pltpu.async_copy(idx_v[q], idx12_dst(ch), so[q])
        pltpu.make_async_copy(idx_v[0], idx12_dst(nc1 - 2), so[0]).wait()
        pltpu.make_async_copy(idx_v[1], idx12_dst(nc1 - 1), so[1]).wait()

        row0w = half * q0

        def pwd_src(ch):
            return pwd_hbm.at[b, pl.ds(row0w + ch * R, R), pl.ds(0, N1)]

        def out_dst(ch):
            return idx01_hbm.at[pl.ds(b * N0 + row0w + ch * R, R)]

        pltpu.async_copy(pwd_src(0), pwd_v[0], sp[0])
        pltpu.async_copy(pwd_src(1), pwd_v[1], sp[1])

        def pair_body(p, carry):
            for q in (0, 1):
                ch = 2 * p + q
                pltpu.make_async_copy(pwd_src(ch), pwd_v[q], sp[q]).wait()

                @pl.when(p > 0)
                def _():
                    pltpu.make_async_copy(idx_v[q], out_dst(ch - 2),
                                          so[q]).wait()
                argmin_chunk(pwd_v[q], N1, idx_v[q])

                @pl.when(ch + 2 < nc0)
                def _():
                    pltpu.async_copy(pwd_src(ch + 2), pwd_v[q], sp[q])
                pltpu.async_copy(idx_v[q], out_dst(ch), so[q])
            return carry

        lax.fori_loop(0, nc0 // 2, pair_body, 0)
        pltpu.make_async_copy(idx_v[0], out_dst(nc0 - 2), so[0]).wait()
        pltpu.make_async_copy(idx_v[1], out_dst(nc0 - 1), so[1]).wait()

    return pl.kernel(
        body,
        out_type=[
            jax.ShapeDtypeStruct((B * N1,), I32),
            jax.ShapeDtypeStruct((B * N0,), I32),
        ],
        mesh=mesh,
        scratch_types=[
            pltpu.VMEM((R, N1), F32),
            pltpu.VMEM((R, N1), F32),
            pltpu.VMEM((R,), I32),
            pltpu.VMEM((R,), I32),
            pltpu.VMEM((L * PITCH,), F32),
            pltpu.VMEM((L * PITCH,), I32),
            pltpu.SemaphoreType.DMA,
            pltpu.SemaphoreType.DMA,
            pltpu.SemaphoreType.DMA,
            pltpu.SemaphoreType.DMA,
        ],
        compiler_params=pltpu.CompilerParams(needs_layout_passes=False),
    )


def _fused_body(x0, x2, x1, i12, i01, W_all, b_all, Wp2a, Wp2b, W2a3, cvec2,
                Wp1a, M1, A1, c1, Wp0a, M0, A0, c0,
                feat2_o, feat1_o, feat0_o):
    x0a = x0[0, :128]
    x0b = x0[0, :512]
    f2 = jnp.dot(x0a, W_all[...], preferred_element_type=F32) + b_all[...]
    cls2 = jnp.max(f2, axis=0, keepdims=True)
    cls_t = jnp.dot(cls2, Wp2a[...], preferred_element_type=F32)
    feat2 = (cls_t
             + jnp.dot(x2[0], W2a3[...], preferred_element_type=F32)
             + jnp.dot(f2.astype(BF16), Wp2b[...].astype(BF16),
                       preferred_element_type=F32)
             + cvec2[...])
    feat2_o[0] = feat2
    T1 = (jnp.dot(feat2.astype(BF16), Wp1a[...].astype(BF16),
                  preferred_element_type=F32)
          - jnp.dot(x2[0], M1[...], preferred_element_type=F32))

    idx12 = i12[0]
    iota2 = lax.broadcasted_iota(I32, (4, 128, 128), 2)
    oh1 = (iota2 == idx12[:, :, None]).astype(F32).reshape(512, 128)
    G1 = jnp.dot(oh1, T1, preferred_element_type=F32)
    feat1 = (G1
             + jnp.dot(x1[0], M1[...], preferred_element_type=F32)
             + jnp.dot(x0b, A1[...], preferred_element_type=F32)
             + c1[...])
    feat1_o[0] = feat1
    T0 = (jnp.dot(feat1.astype(BF16), Wp0a[...].astype(BF16),
                  preferred_element_type=F32)
          - jnp.dot(x1[0], M0[...], preferred_element_type=F32))

    idx01 = i01[0]
    iota0 = lax.broadcasted_iota(I32, (16, 128, 512), 2)
    oh0 = (iota0 == idx01[:, :, None]).astype(BF16).reshape(2048, 512)
    G0 = jnp.dot(oh0, T0.astype(BF16), preferred_element_type=F32)
    feat0_o[0] = (G0
                  + jnp.dot(x0[0], A0[...], preferred_element_type=F32)
                  + c0[...])


def kernel(xyz0, xyz1, xyz2, pwd, W_all, b_all, W2, b2, W1, b1, W0, b0,
           Wp2, bp2, Wp1, bp1, Wp0, bp0):
    B, N0, _ = xyz0.shape
    N1 = xyz1.shape[1]
    N2 = xyz2.shape[1]

    Wp2a, Wp2b = Wp2[:C], Wp2[C:]
    Wp1a, Wp1b = Wp1[:C], Wp1[C:]
    Wp0a, Wp0b = Wp0[:C], Wp0[C:]
    W2a3 = W2 @ Wp2a
    cvec2 = (b2 @ Wp2a + bp2)[None, :]
    M1 = W1 @ Wp1a
    A1 = W_all @ Wp1b
    c1 = (b1 @ Wp1a + b_all @ Wp1b + bp1)[None, :]
    M0 = W0 @ Wp0a
    A0 = M0 + W_all @ Wp0b
    c0 = (b0 @ Wp0a + b_all @ Wp0b + bp0)[None, :]
    b_all2 = b_all[None, :]

    idx12f, idx01f = _make_sc_knn(B, N0, N1, N2)(pwd)
    idx12 = idx12f.reshape(B, N1 // 128, 128)
    idx01 = idx01f.reshape(B, N0 // 128, 128)

    feat2, feat1, feat0 = pl.pallas_call(
        _fused_body,
        grid=(B,),
        in_specs=[
            pl.BlockSpec((1, N0, 3), lambda b: (b, 0, 0)),
            pl.BlockSpec((1, N2, 3), lambda b: (b, 0, 0)),
            pl.BlockSpec((1, N1, 3), lambda b: (b, 0, 0)),
            pl.BlockSpec((1, N1 // 128, 128), lambda b: (b, 0, 0)),
            pl.BlockSpec((1, N0 // 128, 128), lambda b: (b, 0, 0)),
            _full((3, C)), _full((1, C)), _full((C, C)), _full((C, C)),
            _full((3, C)), _full((1, C)), _full((C, C)), _full((3, C)),
            _full((3, C)), _full((1, C)), _full((C, C)), _full((3, C)),
            _full((3, C)), _full((1, C)),
        ],
        out_specs=[
            pl.BlockSpec((1, N2, C), lambda b: (b, 0, 0)),
            pl.BlockSpec((1, N1, C), lambda b: (b, 0, 0)),
            pl.BlockSpec((1, N0, C), lambda b: (b, 0, 0)),
        ],
        out_shape=[
            jax.ShapeDtypeStruct((B, N2, C), F32),
            jax.ShapeDtypeStruct((B, N1, C), F32),
            jax.ShapeDtypeStruct((B, N0, C), F32),
        ],
    )(xyz0, xyz2, xyz1, idx12, idx01, W_all, b_all2, Wp2a, Wp2b, W2a3,
      cvec2, Wp1a, M1, A1, c1, Wp0a, M0, A0, c0)

    return (feat2, feat1, feat0)


# speedup vs baseline: 1.0321x; 1.0039x over previous
import jax
import jax.numpy as jnp
from jax import lax
from jax.experimental import pallas as pl
from jax.experimental.pallas import tpu as pltpu
from jax.experimental.pallas import tpu_sc as plsc

C = 256
F32 = jnp.float32
BF16 = jnp.bfloat16
I32 = jnp.int32

NC = 2
NS = 16
NW = NC * NS
L = 16


def _full(shape):
    return pl.BlockSpec(shape, lambda b: tuple(0 for _ in shape))


def _make_sc_knn(B, N0, N1, N2):
    mesh = plsc.VectorSubcoreMesh(core_axis_name="c", subcore_axis_name="s")
    R = 64
    PITCH = L + 1
    q1 = N1 // 2
    q0 = N0 // 2
    nc1 = q1 // R
    nc0 = q0 // R

    def body(pwd_hbm, idx12_hbm, idx01_hbm,
             pwd_v0, pwd_v1, idx_v0, idx_v1, vbuf, ibuf,
             sp0, sp1, so0, so1):
        cix = lax.axis_index("c")
        six = lax.axis_index("s")
        w = cix * NS + six
        b = w // 2
        half = w % 2
        lane = lax.broadcasted_iota(I32, (L,), 0)
        pwd_v = (pwd_v0, pwd_v1)
        idx_v = (idx_v0, idx_v1)
        sp = (sp0, sp1)
        so = (so0, so1)

        def argmin_chunk(src_ref, k_cols, iv):
            def group_body(g, carry):
                rbase = g * L
                for rr in range(L):
                    r = rbase + rr
                    pairs = []
                    for c16 in range(k_cols // L):
                        v = src_ref[r, pl.ds(c16 * L, L)]
                        pairs.append((v, lane + (c16 * L)))
                    while len(pairs) > 1:
                        nxt = []
                        for k in range(0, len(pairs) - 1, 2):
                            va, ia = pairs[k]
                            vb, ib = pairs[k + 1]
                            mlt = vb < va
                            nxt.append((jnp.where(mlt, vb, va),
                                        jnp.where(mlt, ib, ia)))
                        if len(pairs) % 2:
                            nxt.append(pairs[-1])
                        pairs = nxt
                    v, i = pairs[0]
                    vbuf[pl.ds(rr * PITCH, L)] = v
                    ibuf[pl.ds(rr * PITCH, L)] = i
                col = lane * PITCH
                bv = plsc.load_gather(vbuf, [col])
                bi = plsc.load_gather(ibuf, [col])
                for cc in range(1, L):
                    pv = plsc.load_gather(vbuf, [col + cc])
                    pi = plsc.load_gather(ibuf, [col + cc])
                    better = (pv < bv) | ((pv == bv) & (pi < bi))
                    bv = jnp.where(better, pv, bv)
                    bi = jnp.where(better, pi, bi)
                iv[pl.ds(rbase, L)] = bi
                return carry

            lax.fori_loop(0, R // L, group_body, 0)

        row1 = half * q1

        def pwd12_src(ch):
            return pwd_hbm.at[b, pl.ds(row1 + ch * R, R), pl.ds(0, N2)]

        def pwd12_dst(q):
            return pwd_v[q].at[:, pl.ds(0, N2)]

        def idx12_dst(ch):
            return idx12_hbm.at[pl.ds(b * N1 + row1 + ch * R, R)]

        pltpu.async_copy(pwd12_src(0), pwd12_dst(0), sp0)
        pltpu.async_copy(pwd12_src(1), pwd12_dst(1), sp1)
        for ch in range(nc1):
            q = ch % 2
            pltpu.make_async_copy(pwd12_src(ch), pwd12_dst(q), sp[q]).wait()
            if ch >= 2:
                pltpu.make_async_copy(idx_v[q], idx12_dst(ch - 2),
                                      so[q]).wait()
            argmin_chunk(pwd_v[q], N2, idx_v[q])
            if ch + 2 < nc1:
                pltpu.async_copy(pwd12_src(ch + 2), pwd12_dst(q), sp[q])
            pltpu.async_copy(idx_v[q], idx12_dst(ch), so[q])
        pltpu.make_async_copy(idx_v[0], idx12_dst(nc1 - 2), so[0]).wait()
        pltpu.make_async_copy(idx_v[1], idx12_dst(nc1 - 1), so[1]).wait()

        row0w = half * q0

        def pwd_src(ch):
            return pwd_hbm.at[b, pl.ds(row0w + ch * R, R), pl.ds(0, N1)]

        def out_dst(ch):
            return idx01_hbm.at[pl.ds(b * N0 + row0w + ch * R, R)]

        pltpu.async_copy(pwd_src(0), pwd_v[0], sp[0])
        pltpu.async_copy(pwd_src(1), pwd_v[1], sp[1])

        def pair_body(p, carry):
            for q in (0, 1):
                ch = 2 * p + q
                pltpu.make_async_copy(pwd_src(ch), pwd_v[q], sp[q]).wait()

                @pl.when(p > 0)
                def _():
                    pltpu.make_async_copy(idx_v[q], out_dst(ch - 2),
                                          so[q]).wait()
                argmin_chunk(pwd_v[q], N1, idx_v[q])

                @pl.when(ch + 2 < nc0)
                def _():
                    pltpu.async_copy(pwd_src(ch + 2), pwd_v[q], sp[q])
                pltpu.async_copy(idx_v[q], out_dst(ch), so[q])
            return carry

        lax.fori_loop(0, nc0 // 2, pair_body, 0)
        pltpu.make_async_copy(idx_v[0], out_dst(nc0 - 2), so[0]).wait()
        pltpu.make_async_copy(idx_v[1], out_dst(nc0 - 1), so[1]).wait()

    return pl.kernel(
        body,
        out_type=[
            jax.ShapeDtypeStruct((B * N1,), I32),
            jax.ShapeDtypeStruct((B * N0,), I32),
        ],
        mesh=mesh,
        scratch_types=[
            pltpu.VMEM((R, N1), F32),
            pltpu.VMEM((R, N1), F32),
            pltpu.VMEM((R,), I32),
            pltpu.VMEM((R,), I32),
            pltpu.VMEM((L * PITCH,), F32),
            pltpu.VMEM((L * PITCH,), I32),
            pltpu.SemaphoreType.DMA,
            pltpu.SemaphoreType.DMA,
            pltpu.SemaphoreType.DMA,
            pltpu.SemaphoreType.DMA,
        ],
        compiler_params=pltpu.CompilerParams(needs_layout_passes=False),
    )


def _fused_body(x0, x2, x1, i12, i01, W_all, b_all, Wp2a, Wp2b, W2a3, cvec2,
                Wp1a, M1, A1, c1, Wp0a, M0, A0, c0,
                feat2_o, feat1_o, feat0_o):
    x0a = x0[0, :128]
    x0b = x0[0, :512]
    f2 = jnp.dot(x0a, W_all[...], preferred_element_type=F32) + b_all[...]
    cls2 = jnp.max(f2, axis=0, keepdims=True)
    cls_t = jnp.dot(cls2, Wp2a[...], preferred_element_type=F32)
    feat2 = (cls_t
             + jnp.dot(x2[0], W2a3[...], preferred_element_type=F32)
             + jnp.dot(f2.astype(BF16), Wp2b[...].astype(BF16),
                       preferred_element_type=F32)
             + cvec2[...])
    feat2_o[0] = feat2
    T1 = (jnp.dot(feat2.astype(BF16), Wp1a[...].astype(BF16),
                  preferred_element_type=F32)
          - jnp.dot(x2[0], M1[...], preferred_element_type=F32))

    idx12 = i12[0]
    iota2 = lax.broadcasted_iota(I32, (4, 128, 128), 2)
    oh1 = (iota2 == idx12[:, :, None]).astype(F32).reshape(512, 128)
    G1 = jnp.dot(oh1, T1, preferred_element_type=F32)
    feat1 = (G1
             + jnp.dot(x1[0], M1[...], preferred_element_type=F32)
             + jnp.dot(x0b, A1[...], preferred_element_type=F32)
             + c1[...])
    feat1_o[0] = feat1
    T0 = (jnp.dot(feat1.astype(BF16), Wp0a[...].astype(BF16),
                  preferred_element_type=F32)
          - jnp.dot(x1[0], M0[...], preferred_element_type=F32))

    idx01 = i01[0]
    iota0 = lax.broadcasted_iota(I32, (16, 128, 512), 2)
    oh0 = (iota0 == idx01[:, :, None]).astype(BF16).reshape(2048, 512)
    G0 = jnp.dot(oh0, T0.astype(BF16), preferred_element_type=F32)
    feat0_o[0] = (G0
                  + jnp.dot(x0[0], A0[...], preferred_element_type=F32)
                  + c0[...])


def kernel(xyz0, xyz1, xyz2, pwd, W_all, b_all, W2, b2, W1, b1, W0, b0,
           Wp2, bp2, Wp1, bp1, Wp0, bp0):
    B, N0, _ = xyz0.shape
    N1 = xyz1.shape[1]
    N2 = xyz2.shape[1]

    Wp2a, Wp2b = Wp2[:C], Wp2[C:]
    Wp1a, Wp1b = Wp1[:C], Wp1[C:]
    Wp0a, Wp0b = Wp0[:C], Wp0[C:]
    W2a3 = W2 @ Wp2a
    cvec2 = (b2 @ Wp2a + bp2)[None, :]
    M1 = W1 @ Wp1a
    A1 = W_all @ Wp1b
    c1 = (b1 @ Wp1a + b_all @ Wp1b + bp1)[None, :]
    M0 = W0 @ Wp0a
    A0 = M0 + W_all @ Wp0b
    c0 = (b0 @ Wp0a + b_all @ Wp0b + bp0)[None, :]
    b_all2 = b_all[None, :]

    idx12f, idx01f = _make_sc_knn(B, N0, N1, N2)(pwd)
    idx12 = idx12f.reshape(B, N1 // 128, 128)
    idx01 = idx01f.reshape(B, N0 // 128, 128)

    feat2, feat1, feat0 = pl.pallas_call(
        _fused_body,
        grid=(B,),
        in_specs=[
            pl.BlockSpec((1, N0, 3), lambda b: (b, 0, 0)),
            pl.BlockSpec((1, N2, 3), lambda b: (b, 0, 0)),
            pl.BlockSpec((1, N1, 3), lambda b: (b, 0, 0)),
            pl.BlockSpec((1, N1 // 128, 128), lambda b: (b, 0, 0)),
            pl.BlockSpec((1, N0 // 128, 128), lambda b: (b, 0, 0)),
            _full((3, C)), _full((1, C)), _full((C, C)), _full((C, C)),
            _full((3, C)), _full((1, C)), _full((C, C)), _full((3, C)),
            _full((3, C)), _full((1, C)), _full((C, C)), _full((3, C)),
            _full((3, C)), _full((1, C)),
        ],
        out_specs=[
            pl.BlockSpec((1, N2, C), lambda b: (b, 0, 0)),
            pl.BlockSpec((1, N1, C), lambda b: (b, 0, 0)),
            pl.BlockSpec((1, N0, C), lambda b: (b, 0, 0)),
        ],
        out_shape=[
            jax.ShapeDtypeStruct((B, N2, C), F32),
            jax.ShapeDtypeStruct((B, N1, C), F32),
            jax.ShapeDtypeStruct((B, N0, C), F32),
        ],
    )(xyz0, xyz2, xyz1, idx12, idx01, W_all, b_all2, Wp2a, Wp2b, W2a3,
      cvec2, Wp1a, M1, A1, c1, Wp0a, M0, A0, c0)

    return (feat2, feat1, feat0)
